# Initial kernel scaffold; baseline (speedup 1.0000x reference)
import functools

import jax
import jax.numpy as jnp
from jax import lax
from jax.experimental import pallas as pl
from jax.experimental.pallas import tpu as pltpu
from jax.experimental.pallas import tpu_sc as plsc

N_NODES = 10000
N_FEATS = 128
N_EDGES = 320000

NC = 2
NS = 16
NW = NC * NS

EDGES_PER_W = N_EDGES // NW
CHUNK = 125
NCHUNKS = EDGES_PER_W // CHUNK
ROWS_PER_TILE = N_NODES // NS


def _sc_aggregate_body(hn, src, dst, out, acc, sidx, didx, rows, sem):
    cid = lax.axis_index("c")
    sid = lax.axis_index("s")
    wid = sid * NC + cid

    pltpu.sync_copy(src.at[wid], sidx)
    pltpu.sync_copy(dst.at[wid], didx)

    zeros16 = jnp.zeros((16,), jnp.float32)

    def zrow(r, carry):
        for c8 in range(N_FEATS // 16):
            rows[r, pl.ds(c8 * 16, 16)] = zeros16
        return carry

    lax.fori_loop(0, CHUNK, zrow, 0)
    for k in range(ROWS_PER_TILE // CHUNK):
        pltpu.sync_copy(rows, acc.at[pl.ds(sid * ROWS_PER_TILE + k * CHUNK, CHUNK)])
    plsc.subcore_barrier()

    def step(j, carry):
        pltpu.async_copy(hn.at[sidx.at[j]], rows, sem).wait()
        pltpu.sync_copy(rows, acc.at[didx.at[j]], add=True)
        return carry

    lax.fori_loop(0, NCHUNKS, step, 0)
    plsc.subcore_barrier()

    pltpu.sync_copy(acc.at[pl.ds(sid * ROWS_PER_TILE, ROWS_PER_TILE)],
                    out.at[cid, pl.ds(sid * ROWS_PER_TILE, ROWS_PER_TILE)])


def _make_sc_aggregate(interpret=False):
    return pl.kernel(
        _sc_aggregate_body,
        out_type=jax.ShapeDtypeStruct((NC, N_NODES, N_FEATS), jnp.float32),
        mesh=plsc.VectorSubcoreMesh(core_axis_name="c", subcore_axis_name="s"),
        scratch_types=[
            pltpu.VMEM_SHARED((N_NODES, N_FEATS), jnp.float32),
            pltpu.VMEM((NCHUNKS, CHUNK), jnp.int32),
            pltpu.VMEM((NCHUNKS, CHUNK), jnp.int32),
            pltpu.VMEM((CHUNK, N_FEATS), jnp.float32),
            pltpu.SemaphoreType.DMA,
        ],
        interpret=interpret,
    )


def _scale_body(h_ref, n_ref, o_ref):
    o_ref[...] = h_ref[...] * n_ref[...]


def _final_body(p_ref, n_ref, w_ref, b_ref, o_ref):
    x = (p_ref[0] + p_ref[1]) * n_ref[...]
    o_ref[...] = jnp.dot(x, w_ref[...], preferred_element_type=jnp.float32) + b_ref[...]


_BLK = 1000


def _tc_scale(h, norm2, interpret=False):
    return pl.pallas_call(
        _scale_body,
        grid=(N_NODES // _BLK,),
        in_specs=[
            pl.BlockSpec((_BLK, N_FEATS), lambda i: (i, 0)),
            pl.BlockSpec((_BLK, 1), lambda i: (i, 0)),
        ],
        out_specs=pl.BlockSpec((_BLK, N_FEATS), lambda i: (i, 0)),
        out_shape=jax.ShapeDtypeStruct((N_NODES, N_FEATS), jnp.float32),
        interpret=interpret,
    )(h, norm2)


def _tc_final(parts, norm2, W, b2, interpret=False):
    return pl.pallas_call(
        _final_body,
        grid=(N_NODES // _BLK,),
        in_specs=[
            pl.BlockSpec((NC, _BLK, N_FEATS), lambda i: (0, i, 0)),
            pl.BlockSpec((_BLK, 1), lambda i: (i, 0)),
            pl.BlockSpec((N_FEATS, N_FEATS), lambda i: (0, 0)),
            pl.BlockSpec((1, N_FEATS), lambda i: (0, 0)),
        ],
        out_specs=pl.BlockSpec((_BLK, N_FEATS), lambda i: (i, 0)),
        out_shape=jax.ShapeDtypeStruct((N_NODES, N_FEATS), jnp.float32),
        interpret=interpret,
    )(parts, norm2, W, b2)


def _gcn(h, edge_index, norm, W, b, interpret=False):
    src = edge_index[0].astype(jnp.int32).reshape(NW, NCHUNKS, CHUNK)
    dst = edge_index[1].astype(jnp.int32).reshape(NW, NCHUNKS, CHUNK)
    norm2 = norm.reshape(N_NODES, 1)
    b2 = b.reshape(1, N_FEATS)

    hn = _tc_scale(h, norm2, interpret=interpret)
    parts = _make_sc_aggregate(interpret=interpret)(hn, src, dst)
    return _tc_final(parts, norm2, W, b2, interpret=interpret)


def kernel(h, edge_index, norm, W, b):
    return _gcn(h, edge_index, norm, W, b)


# speedup vs baseline: 8.2009x; 8.2009x over previous
import functools

import jax
import jax.numpy as jnp
from jax import lax
from jax.experimental import pallas as pl
from jax.experimental.pallas import tpu as pltpu
from jax.experimental.pallas import tpu_sc as plsc

N_NODES = 10000
N_FEATS = 128
N_EDGES = 320000

NC = 2
NS = 16
NW = NC * NS

EDGES_PER_W = N_EDGES // NW
CHUNK = 125
NCHUNKS = EDGES_PER_W // CHUNK

ROWS_PER_TILE = 640
N_PAD = NS * ROWS_PER_TILE
ZROWS = 64


def _sc_aggregate_body(hn, src, dst, out, acc, sidx, didx, rows, zbuf, sem):
    cid = lax.axis_index("c")
    sid = lax.axis_index("s")
    wid = sid * NC + cid

    pltpu.sync_copy(src.at[wid], sidx)
    pltpu.sync_copy(dst.at[wid], didx)

    zeros16 = jnp.zeros((16,), jnp.float32)

    def zrow(r, carry):
        for c8 in range(N_FEATS // 16):
            zbuf[r, pl.ds(c8 * 16, 16)] = zeros16
        return carry

    lax.fori_loop(0, ZROWS, zrow, 0)
    for k in range(ROWS_PER_TILE // ZROWS):
        pltpu.sync_copy(zbuf, acc.at[pl.ds(sid * ROWS_PER_TILE + k * ZROWS, ZROWS)])
    plsc.subcore_barrier()

    def step(j, carry):
        pltpu.async_copy(hn.at[sidx.at[j]], rows, sem).wait()
        pltpu.sync_copy(rows, acc.at[didx.at[j]], add=True)
        return carry

    lax.fori_loop(0, NCHUNKS, step, 0)
    plsc.subcore_barrier()

    pltpu.sync_copy(acc.at[pl.ds(sid * ROWS_PER_TILE, ROWS_PER_TILE)],
                    out.at[cid, pl.ds(sid * ROWS_PER_TILE, ROWS_PER_TILE)])


def _make_sc_aggregate(interpret=False):
    return pl.kernel(
        _sc_aggregate_body,
        out_type=jax.ShapeDtypeStruct((NC, N_PAD, N_FEATS), jnp.float32),
        mesh=plsc.VectorSubcoreMesh(core_axis_name="c", subcore_axis_name="s",
                                    num_cores=NC, num_subcores=NS),
        scratch_types=[
            pltpu.VMEM_SHARED((N_PAD, N_FEATS), jnp.float32),
            pltpu.VMEM((NCHUNKS, CHUNK), jnp.int32),
            pltpu.VMEM((NCHUNKS, CHUNK), jnp.int32),
            pltpu.VMEM((CHUNK, N_FEATS), jnp.float32),
            pltpu.VMEM((ZROWS, N_FEATS), jnp.float32),
            pltpu.SemaphoreType.DMA,
        ],
        interpret=interpret,
    )


def _scale_body(h_ref, n_ref, o_ref):
    o_ref[...] = h_ref[...] * n_ref[...]


def _final_body(p_ref, n_ref, w_ref, b_ref, o_ref):
    x = (p_ref[0] + p_ref[1]) * n_ref[...]
    o_ref[...] = jnp.dot(x, w_ref[...], preferred_element_type=jnp.float32) + b_ref[...]


_BLK = 1000


def _tc_scale(h, norm2, interpret=False):
    return pl.pallas_call(
        _scale_body,
        grid=(N_NODES // _BLK,),
        in_specs=[
            pl.BlockSpec((_BLK, N_FEATS), lambda i: (i, 0)),
            pl.BlockSpec((_BLK, 1), lambda i: (i, 0)),
        ],
        out_specs=pl.BlockSpec((_BLK, N_FEATS), lambda i: (i, 0)),
        out_shape=jax.ShapeDtypeStruct((N_NODES, N_FEATS), jnp.float32),
        interpret=interpret,
    )(h, norm2)


def _tc_final(parts, norm2, W, b2, interpret=False):
    return pl.pallas_call(
        _final_body,
        grid=(N_NODES // _BLK,),
        in_specs=[
            pl.BlockSpec((NC, _BLK, N_FEATS), lambda i: (0, i, 0)),
            pl.BlockSpec((_BLK, 1), lambda i: (i, 0)),
            pl.BlockSpec((N_FEATS, N_FEATS), lambda i: (0, 0)),
            pl.BlockSpec((1, N_FEATS), lambda i: (0, 0)),
        ],
        out_specs=pl.BlockSpec((_BLK, N_FEATS), lambda i: (i, 0)),
        out_shape=jax.ShapeDtypeStruct((N_NODES, N_FEATS), jnp.float32),
        interpret=interpret,
    )(parts, norm2, W, b2)


def _gcn(h, edge_index, norm, W, b, interpret=False):
    src = edge_index[0].astype(jnp.int32).reshape(NW, NCHUNKS, CHUNK)
    dst = edge_index[1].astype(jnp.int32).reshape(NW, NCHUNKS, CHUNK)
    norm2 = norm.reshape(N_NODES, 1)
    b2 = b.reshape(1, N_FEATS)

    hn = _tc_scale(h, norm2, interpret=interpret)
    parts = _make_sc_aggregate(interpret=interpret)(hn, src, dst)
    return _tc_final(parts, norm2, W, b2, interpret=interpret)


def kernel(h, edge_index, norm, W, b):
    return _gcn(h, edge_index, norm, W, b)


# speedup vs baseline: 10.9108x; 1.3304x over previous
import functools

import jax
import jax.numpy as jnp
from jax import lax
from jax.experimental import pallas as pl
from jax.experimental.pallas import tpu as pltpu
from jax.experimental.pallas import tpu_sc as plsc

N_NODES = 10000
N_FEATS = 128
N_EDGES = 320000

NC = 2
NS = 16
NW = NC * NS

EDGES_PER_W = N_EDGES // NW
CHUNK = 125
NCHUNKS = EDGES_PER_W // CHUNK

ROWS_PER_TILE = 640
N_PAD = NS * ROWS_PER_TILE
ZROWS = 64


HALF = 16


def _sc_aggregate_body(hn, src, dst, out, acc, sidx, didx, rows0, rows1,
                       zbuf, sg0, sg1):
    cid = lax.axis_index("c")
    sid = lax.axis_index("s")
    wid = sid * NC + cid

    zeros16 = jnp.zeros((16,), jnp.float32)

    def zrow(r, carry):
        for c8 in range(N_FEATS // 16):
            zbuf[r, pl.ds(c8 * 16, 16)] = zeros16
        return carry

    lax.fori_loop(0, ZROWS, zrow, 0)
    for k in range(ROWS_PER_TILE // ZROWS):
        pltpu.sync_copy(zbuf, acc.at[pl.ds(sid * ROWS_PER_TILE + k * ZROWS, ZROWS)])
    plsc.subcore_barrier()

    rows_bufs = (rows0, rows1)
    sems = (sg0, sg1)

    for h in range(NCHUNKS // HALF):
        pltpu.sync_copy(src.at[wid, pl.ds(h * HALF, HALF)], sidx)
        pltpu.sync_copy(dst.at[wid, pl.ds(h * HALF, HALF)], didx)
        pltpu.async_copy(hn.at[sidx.at[0]], rows0, sg0)
        pltpu.async_copy(hn.at[sidx.at[1]], rows1, sg1)

        def pair(p, carry):
            for b in range(2):
                j = p * 2 + b
                pltpu.make_async_copy(hn.at[sidx.at[0]], rows_bufs[b], sems[b]).wait()
                pltpu.sync_copy(rows_bufs[b], acc.at[didx.at[j]], add=True)

                @pl.when(p < HALF // 2 - 1)
                def _():
                    pltpu.async_copy(hn.at[sidx.at[j + 2]], rows_bufs[b], sems[b])
            return carry

        lax.fori_loop(0, HALF // 2, pair, 0)
    plsc.subcore_barrier()

    pltpu.sync_copy(acc.at[pl.ds(sid * ROWS_PER_TILE, ROWS_PER_TILE)],
                    out.at[cid, pl.ds(sid * ROWS_PER_TILE, ROWS_PER_TILE)])


def _make_sc_aggregate(interpret=False):
    return pl.kernel(
        _sc_aggregate_body,
        out_type=jax.ShapeDtypeStruct((NC, N_PAD, N_FEATS), jnp.float32),
        mesh=plsc.VectorSubcoreMesh(core_axis_name="c", subcore_axis_name="s",
                                    num_cores=NC, num_subcores=NS),
        scratch_types=[
            pltpu.VMEM_SHARED((N_PAD, N_FEATS), jnp.float32),
            pltpu.VMEM((HALF, CHUNK), jnp.int32),
            pltpu.VMEM((HALF, CHUNK), jnp.int32),
            pltpu.VMEM((CHUNK, N_FEATS), jnp.float32),
            pltpu.VMEM((CHUNK, N_FEATS), jnp.float32),
            pltpu.VMEM((ZROWS, N_FEATS), jnp.float32),
            pltpu.SemaphoreType.DMA,
            pltpu.SemaphoreType.DMA,
        ],
        interpret=interpret,
    )


def _scale_body(h_ref, n_ref, o_ref):
    o_ref[...] = h_ref[...] * n_ref[...]


def _final_body(p_ref, n_ref, w_ref, b_ref, o_ref):
    x = (p_ref[0] + p_ref[1]) * n_ref[...]
    o_ref[...] = jnp.dot(x, w_ref[...], preferred_element_type=jnp.float32) + b_ref[...]


_BLK = 1000


def _tc_scale(h, norm2, interpret=False):
    return pl.pallas_call(
        _scale_body,
        grid=(N_NODES // _BLK,),
        in_specs=[
            pl.BlockSpec((_BLK, N_FEATS), lambda i: (i, 0)),
            pl.BlockSpec((_BLK, 1), lambda i: (i, 0)),
        ],
        out_specs=pl.BlockSpec((_BLK, N_FEATS), lambda i: (i, 0)),
        out_shape=jax.ShapeDtypeStruct((N_NODES, N_FEATS), jnp.float32),
        interpret=interpret,
    )(h, norm2)


def _tc_final(parts, norm2, W, b2, interpret=False):
    return pl.pallas_call(
        _final_body,
        grid=(N_NODES // _BLK,),
        in_specs=[
            pl.BlockSpec((NC, _BLK, N_FEATS), lambda i: (0, i, 0)),
            pl.BlockSpec((_BLK, 1), lambda i: (i, 0)),
            pl.BlockSpec((N_FEATS, N_FEATS), lambda i: (0, 0)),
            pl.BlockSpec((1, N_FEATS), lambda i: (0, 0)),
        ],
        out_specs=pl.BlockSpec((_BLK, N_FEATS), lambda i: (i, 0)),
        out_shape=jax.ShapeDtypeStruct((N_NODES, N_FEATS), jnp.float32),
        interpret=interpret,
    )(parts, norm2, W, b2)


def _gcn(h, edge_index, norm, W, b, interpret=False):
    src = edge_index[0].astype(jnp.int32).reshape(NW, NCHUNKS, CHUNK)
    dst = edge_index[1].astype(jnp.int32).reshape(NW, NCHUNKS, CHUNK)
    norm2 = norm.reshape(N_NODES, 1)
    b2 = b.reshape(1, N_FEATS)

    hn = _tc_scale(h, norm2, interpret=interpret)
    parts = _make_sc_aggregate(interpret=interpret)(hn, src, dst)
    return _tc_final(parts, norm2, W, b2, interpret=interpret)


def kernel(h, edge_index, norm, W, b):
    return _gcn(h, edge_index, norm, W, b)


# speedup vs baseline: 11.9094x; 1.0915x over previous
import functools

import jax
import jax.numpy as jnp
from jax import lax
from jax.experimental import pallas as pl
from jax.experimental.pallas import tpu as pltpu
from jax.experimental.pallas import tpu_sc as plsc

N_NODES = 10000
N_FEATS = 128
N_EDGES = 320000

NC = 2
NS = 16
NW = NC * NS

EDGES_PER_W = N_EDGES // NW
CHUNK = 125
NCHUNKS = EDGES_PER_W // CHUNK

ROWS_PER_TILE = 640
N_PAD = NS * ROWS_PER_TILE
ZROWS = 64


HALF = 16


def _sc_aggregate_body(hn, edges, out, acc, sidx, didx, rows0, rows1,
                       zbuf, sg0, sg1):
    cid = lax.axis_index("c")
    sid = lax.axis_index("s")
    wid = sid * NC + cid

    zeros16 = jnp.zeros((16,), jnp.float32)

    def zrow(r, carry):
        for c8 in range(N_FEATS // 16):
            zbuf[r, pl.ds(c8 * 16, 16)] = zeros16
        return carry

    lax.fori_loop(0, ZROWS, zrow, 0)
    for k in range(ROWS_PER_TILE // ZROWS):
        pltpu.sync_copy(zbuf, acc.at[pl.ds(sid * ROWS_PER_TILE + k * ZROWS, ZROWS)])
    plsc.subcore_barrier()

    rows_bufs = (rows0, rows1)
    sems = (sg0, sg1)

    for h in range(NCHUNKS // HALF):
        pltpu.sync_copy(edges.at[0, wid, pl.ds(h * HALF, HALF)], sidx)
        pltpu.sync_copy(edges.at[1, wid, pl.ds(h * HALF, HALF)], didx)
        pltpu.async_copy(hn.at[sidx.at[0]], rows0, sg0)
        pltpu.async_copy(hn.at[sidx.at[1]], rows1, sg1)

        def pair(p, carry):
            for b in range(2):
                j = p * 2 + b
                pltpu.make_async_copy(hn.at[sidx.at[0]], rows_bufs[b], sems[b]).wait()
                pltpu.sync_copy(rows_bufs[b], acc.at[didx.at[j]], add=True)

                @pl.when(p < HALF // 2 - 1)
                def _():
                    pltpu.async_copy(hn.at[sidx.at[j + 2]], rows_bufs[b], sems[b])
            return carry

        lax.fori_loop(0, HALF // 2, pair, 0)
    plsc.subcore_barrier()

    pltpu.sync_copy(acc.at[pl.ds(sid * ROWS_PER_TILE, ROWS_PER_TILE)],
                    out.at[cid, pl.ds(sid * ROWS_PER_TILE, ROWS_PER_TILE)])


def _make_sc_aggregate(interpret=False):
    return pl.kernel(
        _sc_aggregate_body,
        out_type=jax.ShapeDtypeStruct((NC, N_PAD, N_FEATS), jnp.float32),
        mesh=plsc.VectorSubcoreMesh(core_axis_name="c", subcore_axis_name="s",
                                    num_cores=NC, num_subcores=NS),
        scratch_types=[
            pltpu.VMEM_SHARED((N_PAD, N_FEATS), jnp.float32),
            pltpu.VMEM((HALF, CHUNK), jnp.int32),
            pltpu.VMEM((HALF, CHUNK), jnp.int32),
            pltpu.VMEM((CHUNK, N_FEATS), jnp.float32),
            pltpu.VMEM((CHUNK, N_FEATS), jnp.float32),
            pltpu.VMEM((ZROWS, N_FEATS), jnp.float32),
            pltpu.SemaphoreType.DMA,
            pltpu.SemaphoreType.DMA,
        ],
        interpret=interpret,
    )


def _scale_body(h_ref, n_ref, o_ref):
    o_ref[...] = h_ref[...] * n_ref[...]


def _final_body(p_ref, n_ref, w_ref, b_ref, o_ref):
    x = (p_ref[0] + p_ref[1]) * n_ref[...]
    o_ref[...] = jnp.dot(x, w_ref[...], preferred_element_type=jnp.float32) + b_ref[...]


_BLK = 2000


def _tc_scale(h, norm2, interpret=False):
    return pl.pallas_call(
        _scale_body,
        grid=(N_NODES // _BLK,),
        in_specs=[
            pl.BlockSpec((_BLK, N_FEATS), lambda i: (i, 0)),
            pl.BlockSpec((_BLK, 1), lambda i: (i, 0)),
        ],
        out_specs=pl.BlockSpec((_BLK, N_FEATS), lambda i: (i, 0)),
        out_shape=jax.ShapeDtypeStruct((N_NODES, N_FEATS), jnp.float32),
        interpret=interpret,
    )(h, norm2)


def _tc_final(parts, norm2, W, b2, interpret=False):
    return pl.pallas_call(
        _final_body,
        grid=(N_NODES // _BLK,),
        in_specs=[
            pl.BlockSpec((NC, _BLK, N_FEATS), lambda i: (0, i, 0)),
            pl.BlockSpec((_BLK, 1), lambda i: (i, 0)),
            pl.BlockSpec((N_FEATS, N_FEATS), lambda i: (0, 0)),
            pl.BlockSpec((1, N_FEATS), lambda i: (0, 0)),
        ],
        out_specs=pl.BlockSpec((_BLK, N_FEATS), lambda i: (i, 0)),
        out_shape=jax.ShapeDtypeStruct((N_NODES, N_FEATS), jnp.float32),
        interpret=interpret,
    )(parts, norm2, W, b2)


def _gcn(h, edge_index, norm, W, b, interpret=False):
    edges = edge_index.astype(jnp.int32).reshape(2, NW, NCHUNKS, CHUNK)
    norm2 = norm.reshape(N_NODES, 1)
    b2 = b.reshape(1, N_FEATS)

    hn = _tc_scale(h, norm2, interpret=interpret)
    parts = _make_sc_aggregate(interpret=interpret)(hn, edges)
    return _tc_final(parts, norm2, W, b2, interpret=interpret)


def kernel(h, edge_index, norm, W, b):
    return _gcn(h, edge_index, norm, W, b)


# speedup vs baseline: 12.9492x; 1.0873x over previous
import jax
import jax.numpy as jnp
from jax import lax
from jax.experimental import pallas as pl
from jax.experimental.pallas import tpu as pltpu
from jax.experimental.pallas import tpu_sc as plsc

N_NODES = 10000
N_FEATS = 128
N_EDGES = 320000

NC = 2
NS = 16
NW = NC * NS

EDGES_PER_W = N_EDGES // NW
CHUNK = 80
NCHUNKS = EDGES_PER_W // CHUNK
NSTAGE = 5
SCH = NCHUNKS // NSTAGE
NBUF = 3

ROWS_PER_TILE = 640
N_PAD = NS * ROWS_PER_TILE
ZROWS = 64


def _sc_aggregate_body(hn, edges, out, acc, sidx, didx, rows0, rows1, rows2,
                       zbuf, sg0, sg1, sg2):
    cid = lax.axis_index("c")
    sid = lax.axis_index("s")
    wid = sid * NC + cid

    zeros16 = jnp.zeros((16,), jnp.float32)

    def zrow(r, carry):
        for c8 in range(N_FEATS // 16):
            zbuf[r, pl.ds(c8 * 16, 16)] = zeros16
        return carry

    lax.fori_loop(0, ZROWS, zrow, 0)
    for k in range(ROWS_PER_TILE // ZROWS):
        pltpu.sync_copy(zbuf, acc.at[pl.ds(sid * ROWS_PER_TILE + k * ZROWS, ZROWS)])
    plsc.subcore_barrier()

    rows_bufs = (rows0, rows1, rows2)
    sems = (sg0, sg1, sg2)

    for st in range(NSTAGE):
        pltpu.sync_copy(edges.at[0, wid, st], sidx)
        pltpu.sync_copy(edges.at[1, wid, st], didx)
        for b in range(NBUF):
            pltpu.async_copy(hn.at[sidx.at[b]], rows_bufs[b], sems[b])

        def group(p, carry):
            for b in range(NBUF):
                j = p * NBUF + b
                pltpu.make_async_copy(hn.at[sidx.at[0]], rows_bufs[b], sems[b]).wait()
                pltpu.sync_copy(rows_bufs[b], acc.at[didx.at[j]], add=True)

                @pl.when(j < SCH - NBUF)
                def _():
                    pltpu.async_copy(hn.at[sidx.at[j + NBUF]], rows_bufs[b], sems[b])
            return carry

        lax.fori_loop(0, (SCH - 1) // NBUF, group, 0)
        jt = SCH - 1
        bt = jt % NBUF
        pltpu.make_async_copy(hn.at[sidx.at[0]], rows_bufs[bt], sems[bt]).wait()
        pltpu.sync_copy(rows_bufs[bt], acc.at[didx.at[jt]], add=True)
    plsc.subcore_barrier()

    pltpu.sync_copy(acc.at[pl.ds(sid * ROWS_PER_TILE, ROWS_PER_TILE)],
                    out.at[cid, pl.ds(sid * ROWS_PER_TILE, ROWS_PER_TILE)])


def _make_sc_aggregate(interpret=False):
    return pl.kernel(
        _sc_aggregate_body,
        out_type=jax.ShapeDtypeStruct((NC, N_PAD, N_FEATS), jnp.float32),
        mesh=plsc.VectorSubcoreMesh(core_axis_name="c", subcore_axis_name="s",
                                    num_cores=NC, num_subcores=NS),
        scratch_types=[
            pltpu.VMEM_SHARED((N_PAD, N_FEATS), jnp.float32),
            pltpu.VMEM((SCH, CHUNK), jnp.int32),
            pltpu.VMEM((SCH, CHUNK), jnp.int32),
            pltpu.VMEM((CHUNK, N_FEATS), jnp.float32),
            pltpu.VMEM((CHUNK, N_FEATS), jnp.float32),
            pltpu.VMEM((CHUNK, N_FEATS), jnp.float32),
            pltpu.VMEM((ZROWS, N_FEATS), jnp.float32),
            pltpu.SemaphoreType.DMA,
            pltpu.SemaphoreType.DMA,
            pltpu.SemaphoreType.DMA,
        ],
        interpret=interpret,
    )


def _scale_body(h_ref, n_ref, o_ref):
    o_ref[...] = h_ref[...] * n_ref[...]


def _final_body(p_ref, n_ref, w_ref, b_ref, o_ref):
    x = (p_ref[0] + p_ref[1]) * n_ref[...]
    o_ref[...] = jnp.dot(x, w_ref[...], preferred_element_type=jnp.float32) + b_ref[...]


_BLK = 2000


def _tc_scale(h, norm2, interpret=False):
    return pl.pallas_call(
        _scale_body,
        grid=(N_NODES // _BLK,),
        in_specs=[
            pl.BlockSpec((_BLK, N_FEATS), lambda i: (i, 0)),
            pl.BlockSpec((_BLK, 1), lambda i: (i, 0)),
        ],
        out_specs=pl.BlockSpec((_BLK, N_FEATS), lambda i: (i, 0)),
        out_shape=jax.ShapeDtypeStruct((N_NODES, N_FEATS), jnp.float32),
        interpret=interpret,
    )(h, norm2)


def _tc_final(parts, norm2, W, b2, interpret=False):
    return pl.pallas_call(
        _final_body,
        grid=(N_NODES // _BLK,),
        in_specs=[
            pl.BlockSpec((NC, _BLK, N_FEATS), lambda i: (0, i, 0)),
            pl.BlockSpec((_BLK, 1), lambda i: (i, 0)),
            pl.BlockSpec((N_FEATS, N_FEATS), lambda i: (0, 0)),
            pl.BlockSpec((1, N_FEATS), lambda i: (0, 0)),
        ],
        out_specs=pl.BlockSpec((_BLK, N_FEATS), lambda i: (i, 0)),
        out_shape=jax.ShapeDtypeStruct((N_NODES, N_FEATS), jnp.float32),
        interpret=interpret,
    )(parts, norm2, W, b2)


def _gcn(h, edge_index, norm, W, b, interpret=False):
    edges = edge_index.astype(jnp.int32).reshape(2, NW, NSTAGE, SCH, CHUNK)
    norm2 = norm.reshape(N_NODES, 1)
    b2 = b.reshape(1, N_FEATS)

    hn = _tc_scale(h, norm2, interpret=interpret)
    parts = _make_sc_aggregate(interpret=interpret)(hn, edges)
    return _tc_final(parts, norm2, W, b2, interpret=interpret)


def kernel(h, edge_index, norm, W, b):
    return _gcn(h, edge_index, norm, W, b)


# speedup vs baseline: 13.0295x; 1.0062x over previous
import jax
import jax.numpy as jnp
from jax import lax
from jax.experimental import pallas as pl
from jax.experimental.pallas import tpu as pltpu
from jax.experimental.pallas import tpu_sc as plsc

N_NODES = 10000
N_FEATS = 128
N_EDGES = 320000

NC = 2
NS = 16
NW = NC * NS

EDGES_PER_W = N_EDGES // NW
CHUNK = 80
NCHUNKS = EDGES_PER_W // CHUNK
NSTAGE = 5
SCH = NCHUNKS // NSTAGE
NBUF = 4

ROWS_PER_TILE = 640
N_PAD = NS * ROWS_PER_TILE
ZROWS = 64


def _sc_aggregate_body(hn, edges, out, acc, sidx, didx, rows0, rows1, rows2,
                       rows3, sg0, sg1, sg2, sg3):
    cid = lax.axis_index("c")
    sid = lax.axis_index("s")
    wid = sid * NC + cid

    zeros16 = jnp.zeros((16,), jnp.float32)

    def zrow(r, carry):
        for c8 in range(N_FEATS // 16):
            rows0[r, pl.ds(c8 * 16, 16)] = zeros16
        return carry

    lax.fori_loop(0, CHUNK, zrow, 0)
    for k in range(ROWS_PER_TILE // CHUNK):
        pltpu.sync_copy(rows0, acc.at[pl.ds(sid * ROWS_PER_TILE + k * CHUNK, CHUNK)])
    plsc.subcore_barrier()

    rows_bufs = (rows0, rows1, rows2, rows3)
    sems = (sg0, sg1, sg2, sg3)

    for st in range(NSTAGE):
        pltpu.sync_copy(edges.at[0, wid, st], sidx)
        pltpu.sync_copy(edges.at[1, wid, st], didx)
        for b in range(NBUF):
            pltpu.async_copy(hn.at[sidx.at[b]], rows_bufs[b], sems[b])

        def group(p, carry):
            for b in range(NBUF):
                j = p * NBUF + b
                pltpu.make_async_copy(hn.at[sidx.at[0]], rows_bufs[b], sems[b]).wait()
                pltpu.sync_copy(rows_bufs[b], acc.at[didx.at[j]], add=True)

                @pl.when(j < SCH - NBUF)
                def _():
                    pltpu.async_copy(hn.at[sidx.at[j + NBUF]], rows_bufs[b], sems[b])
            return carry

        lax.fori_loop(0, (SCH - 1) // NBUF, group, 0)
        jt = SCH - 1
        bt = jt % NBUF
        pltpu.make_async_copy(hn.at[sidx.at[0]], rows_bufs[bt], sems[bt]).wait()
        pltpu.sync_copy(rows_bufs[bt], acc.at[didx.at[jt]], add=True)
    plsc.subcore_barrier()

    pltpu.sync_copy(acc.at[pl.ds(sid * ROWS_PER_TILE, ROWS_PER_TILE)],
                    out.at[cid, pl.ds(sid * ROWS_PER_TILE, ROWS_PER_TILE)])


def _make_sc_aggregate(interpret=False):
    return pl.kernel(
        _sc_aggregate_body,
        out_type=jax.ShapeDtypeStruct((NC, N_PAD, N_FEATS), jnp.float32),
        mesh=plsc.VectorSubcoreMesh(core_axis_name="c", subcore_axis_name="s",
                                    num_cores=NC, num_subcores=NS),
        scratch_types=[
            pltpu.VMEM_SHARED((N_PAD, N_FEATS), jnp.float32),
            pltpu.VMEM((SCH, CHUNK), jnp.int32),
            pltpu.VMEM((SCH, CHUNK), jnp.int32),
            pltpu.VMEM((CHUNK, N_FEATS), jnp.float32),
            pltpu.VMEM((CHUNK, N_FEATS), jnp.float32),
            pltpu.VMEM((CHUNK, N_FEATS), jnp.float32),
            pltpu.VMEM((CHUNK, N_FEATS), jnp.float32),
            pltpu.SemaphoreType.DMA,
            pltpu.SemaphoreType.DMA,
            pltpu.SemaphoreType.DMA,
            pltpu.SemaphoreType.DMA,
        ],
        interpret=interpret,
    )


def _scale_body(h_ref, n_ref, o_ref):
    o_ref[...] = h_ref[...] * n_ref[...]


def _final_body(p_ref, n_ref, w_ref, b_ref, o_ref):
    x = (p_ref[0] + p_ref[1]) * n_ref[...]
    o_ref[...] = jnp.dot(x, w_ref[...], preferred_element_type=jnp.float32) + b_ref[...]


_BLK = 2000


def _tc_scale(h, norm2, interpret=False):
    return pl.pallas_call(
        _scale_body,
        grid=(N_NODES // _BLK,),
        in_specs=[
            pl.BlockSpec((_BLK, N_FEATS), lambda i: (i, 0)),
            pl.BlockSpec((_BLK, 1), lambda i: (i, 0)),
        ],
        out_specs=pl.BlockSpec((_BLK, N_FEATS), lambda i: (i, 0)),
        out_shape=jax.ShapeDtypeStruct((N_NODES, N_FEATS), jnp.float32),
        interpret=interpret,
    )(h, norm2)


def _tc_final(parts, norm2, W, b2, interpret=False):
    return pl.pallas_call(
        _final_body,
        grid=(N_NODES // _BLK,),
        in_specs=[
            pl.BlockSpec((NC, _BLK, N_FEATS), lambda i: (0, i, 0)),
            pl.BlockSpec((_BLK, 1), lambda i: (i, 0)),
            pl.BlockSpec((N_FEATS, N_FEATS), lambda i: (0, 0)),
            pl.BlockSpec((1, N_FEATS), lambda i: (0, 0)),
        ],
        out_specs=pl.BlockSpec((_BLK, N_FEATS), lambda i: (i, 0)),
        out_shape=jax.ShapeDtypeStruct((N_NODES, N_FEATS), jnp.float32),
        interpret=interpret,
    )(parts, norm2, W, b2)


def _gcn(h, edge_index, norm, W, b, interpret=False):
    edges = edge_index.astype(jnp.int32).reshape(2, NW, NSTAGE, SCH, CHUNK)
    norm2 = norm.reshape(N_NODES, 1)
    b2 = b.reshape(1, N_FEATS)

    hn = _tc_scale(h, norm2, interpret=interpret)
    parts = _make_sc_aggregate(interpret=interpret)(hn, edges)
    return _tc_final(parts, norm2, W, b2, interpret=interpret)


def kernel(h, edge_index, norm, W, b):
    return _gcn(h, edge_index, norm, W, b)


# speedup vs baseline: 13.0487x; 1.0015x over previous
import jax
import jax.numpy as jnp
from jax import lax
from jax.experimental import pallas as pl
from jax.experimental.pallas import tpu as pltpu
from jax.experimental.pallas import tpu_sc as plsc

N_NODES = 10000
N_FEATS = 128
N_EDGES = 320000

NC = 2
NS = 16
NW = NC * NS

EDGES_PER_W = N_EDGES // NW
CHUNK = 80
NCHUNKS = EDGES_PER_W // CHUNK
NBUF = 3

ROWS_PER_TILE = 632
N_PAD = NS * ROWS_PER_TILE


def _sc_aggregate_body(hn, pidx_hbm, out, acc, pidx, sring, dring,
                       rows0, rows1, rows2, sg0, sg1, sg2):
    cid = lax.axis_index("c")
    sid = lax.axis_index("s")
    wid = sid * NC + cid

    pltpu.sync_copy(pidx_hbm.at[wid], pidx)

    zeros16 = jnp.zeros((16,), jnp.float32)

    def zrow(r, carry):
        for c8 in range(N_FEATS // 16):
            rows0[r, pl.ds(c8 * 16, 16)] = zeros16
        return carry

    lax.fori_loop(0, CHUNK, zrow, 0)
    base = sid * ROWS_PER_TILE
    for k in range(ROWS_PER_TILE // CHUNK):
        pltpu.sync_copy(rows0, acc.at[pl.ds(base + k * CHUNK, CHUNK)])
    rem = ROWS_PER_TILE % CHUNK
    pltpu.sync_copy(rows0.at[pl.ds(0, rem)],
                    acc.at[pl.ds(base + ROWS_PER_TILE - rem, rem)])
    plsc.subcore_barrier()

    rows_bufs = (rows0, rows1, rows2)
    sems = (sg0, sg1, sg2)
    mask = jnp.full((16,), 0xFFFF, jnp.int32)
    sixteen = jnp.full((16,), 16, jnp.int32)

    def unpack(c, slot):
        for v in range(CHUNK // 16):
            p = pidx[c, pl.ds(v * 16, 16)]
            sring[slot, pl.ds(v * 16, 16)] = p & mask
            dring[slot, pl.ds(v * 16, 16)] = lax.shift_right_logical(p, sixteen)

    for b in range(NBUF):
        unpack(b, b)
        pltpu.async_copy(hn.at[sring.at[b]], rows_bufs[b], sems[b])

    def group(p, carry):
        for b in range(NBUF):
            j = p * NBUF + b
            pltpu.make_async_copy(hn.at[sring.at[0]], rows_bufs[b], sems[b]).wait()
            pltpu.sync_copy(rows_bufs[b], acc.at[dring.at[b]], add=True)

            @pl.when(j < NCHUNKS - NBUF)
            def _():
                unpack(j + NBUF, b)
                pltpu.async_copy(hn.at[sring.at[b]], rows_bufs[b], sems[b])
        return carry

    lax.fori_loop(0, NCHUNKS // NBUF, group, 0)
    for j in range(NBUF * (NCHUNKS // NBUF), NCHUNKS):
        b = j % NBUF
        pltpu.make_async_copy(hn.at[sring.at[0]], rows_bufs[b], sems[b]).wait()
        pltpu.sync_copy(rows_bufs[b], acc.at[dring.at[b]], add=True)
    plsc.subcore_barrier()

    pltpu.sync_copy(acc.at[pl.ds(base, ROWS_PER_TILE)],
                    out.at[cid, pl.ds(base, ROWS_PER_TILE)])


def _make_sc_aggregate(interpret=False):
    return pl.kernel(
        _sc_aggregate_body,
        out_type=jax.ShapeDtypeStruct((NC, N_PAD, N_FEATS), jnp.float32),
        mesh=plsc.VectorSubcoreMesh(core_axis_name="c", subcore_axis_name="s",
                                    num_cores=NC, num_subcores=NS),
        scratch_types=[
            pltpu.VMEM_SHARED((N_PAD, N_FEATS), jnp.float32),
            pltpu.VMEM((NCHUNKS, CHUNK), jnp.int32),
            pltpu.VMEM((NBUF, CHUNK), jnp.int32),
            pltpu.VMEM((NBUF, CHUNK), jnp.int32),
            pltpu.VMEM((CHUNK, N_FEATS), jnp.float32),
            pltpu.VMEM((CHUNK, N_FEATS), jnp.float32),
            pltpu.VMEM((CHUNK, N_FEATS), jnp.float32),
            pltpu.SemaphoreType.DMA,
            pltpu.SemaphoreType.DMA,
            pltpu.SemaphoreType.DMA,
        ],
        interpret=interpret,
    )


def _scale_body(h_ref, n_ref, o_ref):
    o_ref[...] = h_ref[...] * n_ref[...]


def _final_body(p_ref, n_ref, w_ref, b_ref, o_ref):
    x = (p_ref[0] + p_ref[1]) * n_ref[...]
    o_ref[...] = jnp.dot(x, w_ref[...], preferred_element_type=jnp.float32) + b_ref[...]


_BLK = 2000


def _tc_scale(h, norm2, interpret=False):
    return pl.pallas_call(
        _scale_body,
        grid=(N_NODES // _BLK,),
        in_specs=[
            pl.BlockSpec((_BLK, N_FEATS), lambda i: (i, 0)),
            pl.BlockSpec((_BLK, 1), lambda i: (i, 0)),
        ],
        out_specs=pl.BlockSpec((_BLK, N_FEATS), lambda i: (i, 0)),
        out_shape=jax.ShapeDtypeStruct((N_NODES, N_FEATS), jnp.float32),
        interpret=interpret,
    )(h, norm2)


def _tc_final(parts, norm2, W, b2, interpret=False):
    return pl.pallas_call(
        _final_body,
        grid=(N_NODES // _BLK,),
        in_specs=[
            pl.BlockSpec((NC, _BLK, N_FEATS), lambda i: (0, i, 0)),
            pl.BlockSpec((_BLK, 1), lambda i: (i, 0)),
            pl.BlockSpec((N_FEATS, N_FEATS), lambda i: (0, 0)),
            pl.BlockSpec((1, N_FEATS), lambda i: (0, 0)),
        ],
        out_specs=pl.BlockSpec((_BLK, N_FEATS), lambda i: (i, 0)),
        out_shape=jax.ShapeDtypeStruct((N_NODES, N_FEATS), jnp.float32),
        interpret=interpret,
    )(parts, norm2, W, b2)


def _gcn(h, edge_index, norm, W, b, interpret=False):
    src = edge_index[0].astype(jnp.int32)
    dst = edge_index[1].astype(jnp.int32)
    packed = ((dst << 16) | src).reshape(NW, NCHUNKS, CHUNK)
    norm2 = norm.reshape(N_NODES, 1)
    b2 = b.reshape(1, N_FEATS)

    hn = _tc_scale(h, norm2, interpret=interpret)
    parts = _make_sc_aggregate(interpret=interpret)(hn, packed)
    return _tc_final(parts, norm2, W, b2, interpret=interpret)


def kernel(h, edge_index, norm, W, b):
    return _gcn(h, edge_index, norm, W, b)


# speedup vs baseline: 13.0966x; 1.0037x over previous
import jax
import jax.numpy as jnp
from jax import lax
from jax.experimental import pallas as pl
from jax.experimental.pallas import tpu as pltpu
from jax.experimental.pallas import tpu_sc as plsc

N_NODES = 10000
N_FEATS = 128
N_EDGES = 320000

NC = 2
NS = 16
NW = NC * NS

EDGES_PER_W = N_EDGES // NW
CHUNK = 80
NCHUNKS = EDGES_PER_W // CHUNK
NBUF = 3

ROWS_PER_TILE = 632
N_PAD = NS * ROWS_PER_TILE


def _sc_aggregate_body(hn, pidx_hbm, out, acc, pidx, sring, dring,
                       rows0, rows1, rows2, sg0, sg1, sg2):
    cid = lax.axis_index("c")
    sid = lax.axis_index("s")
    wid = sid * NC + cid

    pltpu.sync_copy(pidx_hbm.at[wid], pidx)

    zeros16 = jnp.zeros((16,), jnp.float32)

    def zrow(r, carry):
        for c8 in range(N_FEATS // 16):
            rows0[r, pl.ds(c8 * 16, 16)] = zeros16
        return carry

    lax.fori_loop(0, CHUNK, zrow, 0)
    base = sid * ROWS_PER_TILE
    for k in range(ROWS_PER_TILE // CHUNK):
        pltpu.sync_copy(rows0, acc.at[pl.ds(base + k * CHUNK, CHUNK)])
    rem = ROWS_PER_TILE % CHUNK
    pltpu.sync_copy(rows0.at[pl.ds(0, rem)],
                    acc.at[pl.ds(base + ROWS_PER_TILE - rem, rem)])
    plsc.subcore_barrier()

    rows_bufs = (rows0, rows1, rows2)
    sems = (sg0, sg1, sg2)
    mask = jnp.full((16,), 0xFFFF, jnp.int32)
    sixteen = jnp.full((16,), 16, jnp.int32)

    def unpack(c, slot):
        for v in range(CHUNK // 16):
            p = pidx[c, pl.ds(v * 16, 16)]
            sring[slot, pl.ds(v * 16, 16)] = p & mask
            dring[slot, pl.ds(v * 16, 16)] = lax.shift_right_logical(p, sixteen)

    for b in range(NBUF):
        unpack(b, b)
        pltpu.async_copy(hn.at[sring.at[b]], rows_bufs[b], sems[b])

    def group(p, carry):
        for b in range(NBUF):
            j = p * NBUF + b
            pltpu.make_async_copy(hn.at[sring.at[0]], rows_bufs[b], sems[b]).wait()
            pltpu.sync_copy(rows_bufs[b], acc.at[dring.at[b]], add=True)

            @pl.when(j < NCHUNKS - NBUF)
            def _():
                unpack(j + NBUF, b)
                pltpu.async_copy(hn.at[sring.at[b]], rows_bufs[b], sems[b])
        return carry

    lax.fori_loop(0, NCHUNKS // NBUF, group, 0)
    for j in range(NBUF * (NCHUNKS // NBUF), NCHUNKS):
        b = j % NBUF
        pltpu.make_async_copy(hn.at[sring.at[0]], rows_bufs[b], sems[b]).wait()
        pltpu.sync_copy(rows_bufs[b], acc.at[dring.at[b]], add=True)
    plsc.subcore_barrier()

    pltpu.sync_copy(acc.at[pl.ds(base, ROWS_PER_TILE)],
                    out.at[cid, pl.ds(base, ROWS_PER_TILE)])


def _make_sc_aggregate(interpret=False):
    return pl.kernel(
        _sc_aggregate_body,
        out_type=jax.ShapeDtypeStruct((NC, N_PAD, N_FEATS), jnp.float32),
        mesh=plsc.VectorSubcoreMesh(core_axis_name="c", subcore_axis_name="s",
                                    num_cores=NC, num_subcores=NS),
        scratch_types=[
            pltpu.VMEM_SHARED((N_PAD, N_FEATS), jnp.float32),
            pltpu.VMEM((NCHUNKS, CHUNK), jnp.int32),
            pltpu.VMEM((NBUF, CHUNK), jnp.int32),
            pltpu.VMEM((NBUF, CHUNK), jnp.int32),
            pltpu.VMEM((CHUNK, N_FEATS), jnp.float32),
            pltpu.VMEM((CHUNK, N_FEATS), jnp.float32),
            pltpu.VMEM((CHUNK, N_FEATS), jnp.float32),
            pltpu.SemaphoreType.DMA,
            pltpu.SemaphoreType.DMA,
            pltpu.SemaphoreType.DMA,
        ],
        interpret=interpret,
    )


def _scale_body(h_ref, n_ref, o_ref):
    o_ref[...] = h_ref[...] * n_ref[...]


def _final_body(p_ref, n_ref, w_ref, b_ref, o_ref):
    x = (p_ref[0] + p_ref[1]) * n_ref[...]
    o_ref[...] = jnp.dot(x, w_ref[...], preferred_element_type=jnp.float32) + b_ref[...]


_BLK = 2000


def _tc_scale(h, norm2, interpret=False):
    return pl.pallas_call(
        _scale_body,
        grid=(N_NODES // _BLK,),
        in_specs=[
            pl.BlockSpec((_BLK, N_FEATS), lambda i: (i, 0)),
            pl.BlockSpec((_BLK, 1), lambda i: (i, 0)),
        ],
        out_specs=pl.BlockSpec((_BLK, N_FEATS), lambda i: (i, 0)),
        out_shape=jax.ShapeDtypeStruct((N_NODES, N_FEATS), jnp.float32),
        interpret=interpret,
    )(h, norm2)


def _tc_final(parts, norm2, W, b2, interpret=False):
    return pl.pallas_call(
        _final_body,
        grid=(N_NODES // _BLK,),
        in_specs=[
            pl.BlockSpec((NC, _BLK, N_FEATS), lambda i: (0, i, 0)),
            pl.BlockSpec((_BLK, 1), lambda i: (i, 0)),
            pl.BlockSpec((N_FEATS, N_FEATS), lambda i: (0, 0)),
            pl.BlockSpec((1, N_FEATS), lambda i: (0, 0)),
        ],
        out_specs=pl.BlockSpec((_BLK, N_FEATS), lambda i: (i, 0)),
        out_shape=jax.ShapeDtypeStruct((N_NODES, N_FEATS), jnp.float32),
        interpret=interpret,
    )(parts, norm2, W, b2)


def _gcn(h, edge_index, norm, W, b, interpret=False):
    e32 = edge_index.astype(jnp.int32)
    weights = jnp.array([[1], [65536]], jnp.int32)
    packed = (e32 * weights).sum(axis=0).reshape(NW, NCHUNKS, CHUNK)
    norm2 = norm.reshape(N_NODES, 1)
    b2 = b.reshape(1, N_FEATS)

    hn = _tc_scale(h, norm2, interpret=interpret)
    parts = _make_sc_aggregate(interpret=interpret)(hn, packed)
    return _tc_final(parts, norm2, W, b2, interpret=interpret)


def kernel(h, edge_index, norm, W, b):
    return _gcn(h, edge_index, norm, W, b)


# speedup vs baseline: 13.1906x; 1.0072x over previous
import jax
import jax.numpy as jnp
from jax import lax
from jax.experimental import pallas as pl
from jax.experimental.pallas import tpu as pltpu
from jax.experimental.pallas import tpu_sc as plsc

N_NODES = 10000
N_FEATS = 128
N_EDGES = 320000

NC = 2
NS = 16
NW = NC * NS

EDGES_PER_W = N_EDGES // NW
CHUNK = 80
NCHUNKS = EDGES_PER_W // CHUNK
NBUF = 3

ROWS_PER_TILE = 632
N_PAD = NS * ROWS_PER_TILE


def _sc_aggregate_body(hn, pidx_hbm, out, acc, pidx, sring, dring,
                       rows0, rows1, rows2, sg0, sg1, sg2):
    cid = lax.axis_index("c")
    sid = lax.axis_index("s")
    wid = sid * NC + cid

    pltpu.sync_copy(pidx_hbm.at[wid], pidx)

    rows_bufs = (rows0, rows1, rows2)
    sems = (sg0, sg1, sg2)
    mask = jnp.full((16,), 0xFFFF, jnp.int32)
    sixteen = jnp.full((16,), 16, jnp.int32)

    def unpack(c, slot):
        for v in range(CHUNK // 16):
            p = pidx[c, pl.ds(v * 16, 16)]
            sring[slot, pl.ds(v * 16, 16)] = p & mask
            dring[slot, pl.ds(v * 16, 16)] = lax.shift_right_logical(p, sixteen)

    for b in range(NBUF):
        unpack(b, b)
    for b in range(1, NBUF):
        pltpu.async_copy(hn.at[sring.at[b]], rows_bufs[b], sems[b])

    zeros16 = jnp.zeros((16,), jnp.float32)

    def zrow(r, carry):
        for c8 in range(N_FEATS // 16):
            rows0[r, pl.ds(c8 * 16, 16)] = zeros16
        return carry

    lax.fori_loop(0, CHUNK, zrow, 0)
    base = sid * ROWS_PER_TILE
    for k in range(ROWS_PER_TILE // CHUNK):
        pltpu.sync_copy(rows0, acc.at[pl.ds(base + k * CHUNK, CHUNK)])
    rem = ROWS_PER_TILE % CHUNK
    pltpu.sync_copy(rows0.at[pl.ds(0, rem)],
                    acc.at[pl.ds(base + ROWS_PER_TILE - rem, rem)])
    pltpu.async_copy(hn.at[sring.at[0]], rows_bufs[0], sems[0])
    plsc.subcore_barrier()

    def group(p, carry):
        for b in range(NBUF):
            j = p * NBUF + b
            pltpu.make_async_copy(hn.at[sring.at[0]], rows_bufs[b], sems[b]).wait()
            pltpu.sync_copy(rows_bufs[b], acc.at[dring.at[b]], add=True)

            @pl.when(j < NCHUNKS - NBUF)
            def _():
                unpack(j + NBUF, b)
                pltpu.async_copy(hn.at[sring.at[b]], rows_bufs[b], sems[b])
        return carry

    lax.fori_loop(0, NCHUNKS // NBUF, group, 0)
    for j in range(NBUF * (NCHUNKS // NBUF), NCHUNKS):
        b = j % NBUF
        pltpu.make_async_copy(hn.at[sring.at[0]], rows_bufs[b], sems[b]).wait()
        pltpu.sync_copy(rows_bufs[b], acc.at[dring.at[b]], add=True)
    plsc.subcore_barrier()

    pltpu.sync_copy(acc.at[pl.ds(base, ROWS_PER_TILE)],
                    out.at[cid, pl.ds(base, ROWS_PER_TILE)])


def _make_sc_aggregate(interpret=False):
    return pl.kernel(
        _sc_aggregate_body,
        out_type=jax.ShapeDtypeStruct((NC, N_PAD, N_FEATS), jnp.float32),
        mesh=plsc.VectorSubcoreMesh(core_axis_name="c", subcore_axis_name="s",
                                    num_cores=NC, num_subcores=NS),
        scratch_types=[
            pltpu.VMEM_SHARED((N_PAD, N_FEATS), jnp.float32),
            pltpu.VMEM((NCHUNKS, CHUNK), jnp.int32),
            pltpu.VMEM((NBUF, CHUNK), jnp.int32),
            pltpu.VMEM((NBUF, CHUNK), jnp.int32),
            pltpu.VMEM((CHUNK, N_FEATS), jnp.float32),
            pltpu.VMEM((CHUNK, N_FEATS), jnp.float32),
            pltpu.VMEM((CHUNK, N_FEATS), jnp.float32),
            pltpu.SemaphoreType.DMA,
            pltpu.SemaphoreType.DMA,
            pltpu.SemaphoreType.DMA,
        ],
        interpret=interpret,
    )


def _scale_body(h_ref, n_ref, o_ref):
    o_ref[...] = h_ref[...] * n_ref[...]


def _final_body(p_ref, n_ref, w_ref, b_ref, o_ref):
    x = (p_ref[0] + p_ref[1]) * n_ref[...]
    o_ref[...] = jnp.dot(x, w_ref[...], preferred_element_type=jnp.float32) + b_ref[...]


_BLK = 2000


def _tc_scale(h, norm2, interpret=False):
    return pl.pallas_call(
        _scale_body,
        grid=(N_NODES // _BLK,),
        in_specs=[
            pl.BlockSpec((_BLK, N_FEATS), lambda i: (i, 0)),
            pl.BlockSpec((_BLK, 1), lambda i: (i, 0)),
        ],
        out_specs=pl.BlockSpec((_BLK, N_FEATS), lambda i: (i, 0)),
        out_shape=jax.ShapeDtypeStruct((N_NODES, N_FEATS), jnp.float32),
        interpret=interpret,
    )(h, norm2)


def _tc_final(parts, norm2, W, b2, interpret=False):
    return pl.pallas_call(
        _final_body,
        grid=(N_NODES // _BLK,),
        in_specs=[
            pl.BlockSpec((NC, _BLK, N_FEATS), lambda i: (0, i, 0)),
            pl.BlockSpec((_BLK, 1), lambda i: (i, 0)),
            pl.BlockSpec((N_FEATS, N_FEATS), lambda i: (0, 0)),
            pl.BlockSpec((1, N_FEATS), lambda i: (0, 0)),
        ],
        out_specs=pl.BlockSpec((_BLK, N_FEATS), lambda i: (i, 0)),
        out_shape=jax.ShapeDtypeStruct((N_NODES, N_FEATS), jnp.float32),
        interpret=interpret,
    )(parts, norm2, W, b2)


def _gcn(h, edge_index, norm, W, b, interpret=False):
    e32 = edge_index.astype(jnp.int32)
    weights = jnp.array([[1], [65536]], jnp.int32)
    packed = (e32 * weights).sum(axis=0).reshape(NW, NCHUNKS, CHUNK)
    norm2 = norm.reshape(N_NODES, 1)
    b2 = b.reshape(1, N_FEATS)

    hn = _tc_scale(h, norm2, interpret=interpret)
    parts = _make_sc_aggregate(interpret=interpret)(hn, packed)
    return _tc_final(parts, norm2, W, b2, interpret=interpret)


def kernel(h, edge_index, norm, W, b):
    return _gcn(h, edge_index, norm, W, b)
